# baseline (speedup 1.0000x reference)
import functools

import jax
import jax.numpy as jnp
from jax import lax
from jax.experimental import pallas as pl
from jax.experimental.pallas import tpu as pltpu
from jax.experimental.pallas import tpu_sc as plsc

N_NODES = 10000
D = 128
N_EDGES = 320000
EPS = 1e-10

NC = 2
NS = 16
L = 16

N_TILES = NC * NS
EDGES_PER_TILE = N_EDGES // N_TILES
CHUNK = 80
PH_C = 25
PH_E = PH_C * CHUNK
N_PH = EDGES_PER_TILE // PH_E
N_PAD = 10240
ROWS_PER_TILE = N_PAD // NS


def _sc_graph_conv(sidx, tidx3d, enorm, emb, zeros_init):
  mesh = plsc.VectorSubcoreMesh(core_axis_name="c", subcore_axis_name="s")

  @functools.partial(
      pl.kernel,
      mesh=mesh,
      out_type=jax.ShapeDtypeStruct((NC * N_PAD, D), jnp.float32),
      scratch_types=[
          pltpu.VMEM((PH_E,), jnp.int32),
          pltpu.VMEM((PH_C, CHUNK), jnp.int32),
          pltpu.VMEM((PH_E,), jnp.float32),
          pltpu.VMEM((CHUNK, D), jnp.float32),
          pltpu.VMEM((CHUNK, D), jnp.float32),
          pltpu.SemaphoreType.DMA,
          pltpu.SemaphoreType.DMA,
          pltpu.SemaphoreType.DMA,
          pltpu.VMEM_SHARED((N_PAD, D), jnp.float32),
      ],
  )
  def k(sidx_hbm, tidx_hbm, en_hbm, emb_hbm, zeros_hbm, out_hbm,
        sidx_v, tidx_v, en_v, rows0, rows1, sem_ph, sem_r0, sem_r1, acc):
    c = lax.axis_index("c")
    s = lax.axis_index("s")
    rows = (rows0, rows1)
    sems = (sem_r0, sem_r1)

    pltpu.sync_copy(zeros_hbm, acc.at[pl.ds(s * ROWS_PER_TILE, ROWS_PER_TILE)])
    plsc.subcore_barrier()

    tile_id = c * NS + s
    tile_base = tile_id * EDGES_PER_TILE

    def start_gather(ci, b):
      pltpu.async_copy(
          emb_hbm.at[sidx_v.at[pl.ds(ci * CHUNK, CHUNK)]], rows[b], sems[b])

    def wait_gather(b):
      pltpu.make_async_copy(emb_hbm.at[pl.ds(0, CHUNK)], rows[b],
                            sems[b]).wait()

    def process(ci, b):
      wait_gather(b)
      rv = rows[b]

      def group_body(eb, _):
        en16 = en_v[pl.ds(ci * CHUNK + eb * L, L)]
        for j in range(L):
          e = eb * L + j
          en = jnp.full((L,), en16[j], dtype=jnp.float32)
          for g in range(D // L):
            sl = pl.ds(g * L, L)
            rv[e, sl] = rv[e, sl] * en
        return 0

      lax.fori_loop(0, CHUNK // L, group_body, 0)

      pltpu.sync_copy(rv, acc.at[tidx_v.at[ci]], add=True)

    def phase_body(p, _):
      base = tile_base + p * PH_E
      cp1 = pltpu.async_copy(sidx_hbm.at[pl.ds(base, PH_E)], sidx_v, sem_ph)
      cp2 = pltpu.async_copy(tidx_hbm.at[tile_id * N_PH + p], tidx_v, sem_ph)
      cp3 = pltpu.async_copy(en_hbm.at[pl.ds(base, PH_E)], en_v, sem_ph)
      cp1.wait()
      cp2.wait()
      cp3.wait()

      start_gather(0, 0)

      def pair_body(kk, _):
        a = 2 * kk
        start_gather(a + 1, 1)
        process(a, 0)
        start_gather(a + 2, 0)
        process(a + 1, 1)
        return 0

      lax.fori_loop(0, (PH_C - 1) // 2, pair_body, 0)
      process(PH_C - 1, 0)
      return 0

    lax.fori_loop(0, N_PH, phase_body, 0)
    plsc.subcore_barrier()

    row0 = s * ROWS_PER_TILE
    pltpu.sync_copy(acc.at[pl.ds(row0, ROWS_PER_TILE)],
                    out_hbm.at[pl.ds(c * N_PAD + row0, ROWS_PER_TILE)])

  return k(sidx, tidx3d, enorm, emb, zeros_init)


ROW_BLK = 1000


def _tc_head_body(part_ref, wl_ref, bl_ref, ws_ref, bs_ref, loc_ref, std_ref):
  p = part_ref[0] + part_ref[1]
  dn = (((1,), (1,)), ((), ()))
  loc = lax.dot_general(p, wl_ref[...], dn,
                        preferred_element_type=jnp.float32)
  loc_ref[...] = loc + bl_ref[...]
  z = lax.dot_general(p, ws_ref[...], dn,
                      preferred_element_type=jnp.float32) + bs_ref[...]
  std_ref[...] = jnp.logaddexp(z, 0.0) + EPS


def _tc_head(partials, W_loc, b_loc, W_std, b_std):
  grid = (N_NODES // ROW_BLK,)
  return pl.pallas_call(
      _tc_head_body,
      grid=grid,
      in_specs=[
          pl.BlockSpec((NC, ROW_BLK, D), lambda i: (0, i, 0)),
          pl.BlockSpec((D, D), lambda i: (0, 0)),
          pl.BlockSpec((1, D), lambda i: (0, 0)),
          pl.BlockSpec((D, D), lambda i: (0, 0)),
          pl.BlockSpec((1, D), lambda i: (0, 0)),
      ],
      out_specs=[
          pl.BlockSpec((ROW_BLK, D), lambda i: (i, 0)),
          pl.BlockSpec((ROW_BLK, D), lambda i: (i, 0)),
      ],
      out_shape=[
          jax.ShapeDtypeStruct((N_NODES, D), jnp.float32),
          jax.ShapeDtypeStruct((N_NODES, D), jnp.float32),
      ],
  )(partials, W_loc, b_loc, W_std, b_std)


def kernel(edge_index, edge_norm, emb, W_loc, b_loc, W_std, b_std):
  sidx = edge_index[0]
  tidx3d = edge_index[1].reshape(N_TILES * N_PH, PH_C, CHUNK)
  zeros_init = jnp.zeros((ROWS_PER_TILE, D), jnp.float32)
  flat = _sc_graph_conv(sidx, tidx3d, edge_norm, emb, zeros_init)
  partials = flat.reshape(NC, N_PAD, D)[:, :N_NODES]
  loc, std = _tc_head(partials, W_loc, b_loc.reshape(1, D),
                      W_std, b_std.reshape(1, D))
  return (loc, std)
